# Initial kernel scaffold; baseline (speedup 1.0000x reference)
import jax
import jax.numpy as jnp
from jax.experimental import pallas as pl


def kernel(bus_type, embd_table):
    raise NotImplementedError("write your pallas kernel here")


# speedup vs baseline: 1.4055x; 1.4055x over previous
import functools

import jax
import jax.numpy as jnp
from jax import lax
from jax.experimental import pallas as pl
from jax.experimental.pallas import tpu as pltpu
from jax.experimental.pallas import tpu_sc as plsc

BATCH = 16384
EMBD_DIM = 32
NUM_CORES = 2
NUM_SUBCORES = 16
NUM_WORKERS = NUM_CORES * NUM_SUBCORES
B_PER_W = BATCH // NUM_WORKERS
LANES = 16
N_GROUPS = B_PER_W // LANES
SLAB = B_PER_W * EMBD_DIM


def _lookup_body(idx_hbm, table_hbm, out_hbm, idx_v, tab_v, out_v):
    wid = lax.axis_index("s") * NUM_CORES + lax.axis_index("c")
    pltpu.sync_copy(table_hbm, tab_v)
    pltpu.sync_copy(idx_hbm.at[pl.ds(wid * B_PER_W, B_PER_W)], idx_v)

    lane = lax.iota(jnp.int32, LANES)
    row_off = lane * EMBD_DIM

    def group(g, carry):
        vidx = idx_v[pl.ds(pl.multiple_of(g * LANES, LANES), LANES)]
        src_base = vidx * EMBD_DIM
        dst_base = row_off + g * (LANES * EMBD_DIM)
        for d in range(EMBD_DIM):
            vals = plsc.load_gather(tab_v, [src_base + d])
            plsc.store_scatter(out_v, [dst_base + d], vals)
        return carry

    lax.fori_loop(0, N_GROUPS, group, 0)
    pltpu.sync_copy(out_v, out_hbm.at[pl.ds(wid * SLAB, SLAB)])


@jax.jit
def _lookup(idx_flat, table_flat):
    mesh = plsc.VectorSubcoreMesh(core_axis_name="c", subcore_axis_name="s")
    return pl.kernel(
        _lookup_body,
        out_type=jax.ShapeDtypeStruct((BATCH * EMBD_DIM,), jnp.float32),
        mesh=mesh,
        compiler_params=pltpu.CompilerParams(needs_layout_passes=False),
        scratch_types=[
            pltpu.VMEM((B_PER_W,), jnp.int32),
            pltpu.VMEM((3 * EMBD_DIM,), jnp.float32),
            pltpu.VMEM((SLAB,), jnp.float32),
        ],
    )(idx_flat, table_flat)


def kernel(bus_type, embd_table):
    idx_flat = bus_type.astype(jnp.int32).reshape(BATCH)
    out = _lookup(idx_flat, embd_table.reshape(-1))
    return out.reshape(BATCH, EMBD_DIM)


# speedup vs baseline: 1.4922x; 1.0617x over previous
import functools

import jax
import jax.numpy as jnp
from jax import lax
from jax.experimental import pallas as pl
from jax.experimental.pallas import tpu as pltpu
from jax.experimental.pallas import tpu_sc as plsc

BATCH = 16384
EMBD_DIM = 32
NUM_CORES = 2
NUM_SUBCORES = 16
NUM_WORKERS = NUM_CORES * NUM_SUBCORES
B_PER_W = BATCH // NUM_WORKERS
LANES = 16
N_GROUPS = B_PER_W // LANES
SLAB = B_PER_W * EMBD_DIM


def _lookup_body(idx_hbm, table_hbm, out_hbm, idx_v, tab_v, out_v):
    wid = lax.axis_index("s") * NUM_CORES + lax.axis_index("c")
    pltpu.sync_copy(table_hbm, tab_v)
    pltpu.sync_copy(idx_hbm.at[pl.ds(wid * B_PER_W, B_PER_W)], idx_v)

    lane = lax.iota(jnp.int32, LANES)
    row_off = lane * EMBD_DIM

    @plsc.parallel_loop(0, N_GROUPS, unroll=2)
    def _group(g):
        vidx = idx_v[pl.ds(pl.multiple_of(g * LANES, LANES), LANES)]
        src_base = vidx * EMBD_DIM
        dst_base = row_off + g * (LANES * EMBD_DIM)
        vals = [plsc.load_gather(tab_v, [src_base + d]) for d in range(EMBD_DIM)]
        for d in range(EMBD_DIM):
            plsc.store_scatter(out_v, [dst_base + d], vals[d])
    pltpu.sync_copy(out_v, out_hbm.at[pl.ds(wid * SLAB, SLAB)])


@jax.jit
def _lookup(idx_flat, table_flat):
    mesh = plsc.VectorSubcoreMesh(core_axis_name="c", subcore_axis_name="s")
    return pl.kernel(
        _lookup_body,
        out_type=jax.ShapeDtypeStruct((BATCH * EMBD_DIM,), jnp.float32),
        mesh=mesh,
        compiler_params=pltpu.CompilerParams(needs_layout_passes=False),
        scratch_types=[
            pltpu.VMEM((B_PER_W,), jnp.int32),
            pltpu.VMEM((3 * EMBD_DIM,), jnp.float32),
            pltpu.VMEM((SLAB,), jnp.float32),
        ],
    )(idx_flat, table_flat)


def kernel(bus_type, embd_table):
    idx_flat = bus_type.astype(jnp.int32).reshape(BATCH)
    out = _lookup(idx_flat, embd_table.reshape(-1))
    return out.reshape(BATCH, EMBD_DIM)


# speedup vs baseline: 2.3446x; 1.5712x over previous
import functools

import jax
import jax.numpy as jnp
from jax import lax
from jax.experimental import pallas as pl
from jax.experimental.pallas import tpu as pltpu
from jax.experimental.pallas import tpu_sc as plsc

BATCH = 16384
EMBD_DIM = 32
NUM_CORES = 2
NUM_SUBCORES = 16
NUM_WORKERS = NUM_CORES * NUM_SUBCORES
B_PER_W = BATCH // NUM_WORKERS
LANES = 16


def _lookup_body(idx_hbm, table_hbm, out_hbm, idx_v, tab_v, out_v):
    wid = lax.axis_index("s") * NUM_CORES + lax.axis_index("c")
    pltpu.sync_copy(table_hbm, tab_v)
    pltpu.sync_copy(idx_hbm.at[pl.ds(wid * B_PER_W, B_PER_W)], idx_v)

    t0a = tab_v[pl.ds(0, LANES)]
    t0b = tab_v[pl.ds(16, LANES)]
    d10a = tab_v[pl.ds(32, LANES)] - t0a
    d10b = tab_v[pl.ds(48, LANES)] - t0b
    d20a = tab_v[pl.ds(64, LANES)] - t0a
    d20b = tab_v[pl.ds(80, LANES)] - t0b

    @plsc.parallel_loop(0, B_PER_W // LANES, unroll=2)
    def _group(g):
        base = pl.multiple_of(g * LANES, LANES)
        vidx = idx_v[pl.ds(base, LANES)]
        vf1 = (vidx == 1).astype(jnp.float32)
        vf2 = (vidx == 2).astype(jnp.float32)
        for j in range(LANES):
            f1 = vf1[j]
            f2 = vf2[j]
            out_v[base + j, pl.ds(0, LANES)] = t0a + f1 * d10a + f2 * d20a
            out_v[base + j, pl.ds(LANES, LANES)] = t0b + f1 * d10b + f2 * d20b

    pltpu.sync_copy(out_v, out_hbm.at[pl.ds(wid * B_PER_W, B_PER_W)])


@jax.jit
def _lookup(idx_flat, table_flat):
    mesh = plsc.VectorSubcoreMesh(core_axis_name="c", subcore_axis_name="s")
    return pl.kernel(
        _lookup_body,
        out_type=jax.ShapeDtypeStruct((BATCH, EMBD_DIM), jnp.float32),
        mesh=mesh,
        compiler_params=pltpu.CompilerParams(needs_layout_passes=False),
        scratch_types=[
            pltpu.VMEM((B_PER_W,), jnp.int32),
            pltpu.VMEM((3 * EMBD_DIM,), jnp.float32),
            pltpu.VMEM((B_PER_W, EMBD_DIM), jnp.float32),
        ],
    )(idx_flat, table_flat)


def kernel(bus_type, embd_table):
    idx_flat = bus_type.astype(jnp.int32).reshape(BATCH)
    return _lookup(idx_flat, embd_table.reshape(-1))
